# baseline (speedup 1.0000x reference)
import functools

import jax
import jax.numpy as jnp
from jax.experimental import pallas as pl
from jax.experimental.pallas import tpu as pltpu
from jax.experimental.pallas import tpu_sc as plsc

_T = 2048
_D = 1024
_E = 8
_TB = 512
_NT = 16
_XS = _NT * _TB
_NA = 2 * _T
_NC = 2
_NS = 16
_NW = _NC * _NS
_CHUNK = 64
_PER_W = (2 * _NA) // _NW
_NCHUNK = _PER_W // _CHUNK


def _shift_down(a, s):
    return jnp.concatenate([jnp.zeros((s, a.shape[1]), a.dtype), a[: a.shape[0] - s]], axis=0)


def _lane_shift(a, s):
    return jnp.concatenate([jnp.zeros((a.shape[0], s), a.dtype), a[:, : a.shape[1] - s]], axis=1)


def _route_kernel(x_ref, wg_ref, g_ref, p0_ref, p1_ref, meta_ref):
    task = pl.program_id(0)
    x = x_ref[0]
    wg = wg_ref[0]
    logits = jax.lax.dot_general(
        x, wg, (((1,), (1,)), ((), ())), preferred_element_type=jnp.float32
    )
    iota = jax.lax.broadcasted_iota(jnp.int32, logits.shape, 1)
    m1 = jnp.max(logits, axis=1, keepdims=True)
    i1 = jnp.min(jnp.where(logits == m1, iota, _E), axis=1, keepdims=True)
    mask1 = iota == i1
    rest = jnp.where(mask1, -jnp.inf, logits)
    m2 = jnp.max(rest, axis=1, keepdims=True)
    i2 = jnp.min(jnp.where(rest == m2, iota, _E), axis=1, keepdims=True)
    mask2 = iota == i2
    tm = jnp.where(mask1 | mask2, logits, 0.0)
    gates = tm / (jnp.sum(tm, axis=1, keepdims=True) + 1e-9)
    g0 = jnp.sum(jnp.where(mask1, gates, 0.0), axis=1, keepdims=True)
    g1 = jnp.sum(jnp.where(mask2, gates, 0.0), axis=1, keepdims=True)
    g_ref[0] = jnp.where(iota == 0, g0, 0.0) + jnp.where(iota == 1, g1, 0.0)

    oh0 = mask1.astype(jnp.float32)
    oh1 = mask2.astype(jnp.float32)
    c0i = oh0
    c1i = oh1
    s = 1
    while s < _T:
        c0i = c0i + _shift_down(c0i, s)
        c1i = c1i + _shift_down(c1i, s)
        s *= 2
    cum0 = c0i - oh0
    cum1 = c1i - oh1
    cnt0 = jnp.max(c0i, axis=0, keepdims=True)
    cnt1 = jnp.max(c1i, axis=0, keepdims=True)
    n_e = cnt0 + cnt1
    tiles = jnp.floor((n_e + (_TB - 1)) * (1.0 / _TB))
    ti = tiles
    for sh in (1, 2, 4):
        ti = ti + _lane_shift(ti, sh)
    tile_start = ti - tiles
    total_tiles = ti[:, _E - 1 : _E]
    po = tile_start * float(_TB)

    task_off = (task * _XS).astype(jnp.float32)
    pos0 = jnp.sum(oh0 * (po + cum0), axis=1, keepdims=True) + task_off
    pos1 = jnp.sum(oh1 * (po + cnt0 + cum1), axis=1, keepdims=True) + task_off
    ones8 = jnp.ones((1, _E), jnp.float32)
    p0_ref[0] = (pos0 * ones8).astype(jnp.int32)
    p1_ref[0] = (pos1 * ones8).astype(jnp.int32)

    ts16 = jnp.broadcast_to(tile_start, (_NT, _E))
    jvec = jax.lax.broadcasted_iota(jnp.int32, (_NT, _E), 0).astype(jnp.float32)
    te = jnp.sum((ts16 <= jvec).astype(jnp.float32), axis=1, keepdims=True) - 1.0
    te = jnp.clip(te, 0.0, float(_E - 1))
    teb = jnp.broadcast_to(te, (_NT, 128))
    ttb = jnp.broadcast_to(total_tiles, (_NT, 128))
    meta_ref[0] = jnp.concatenate([teb, ttb], axis=0).astype(jnp.int32)


def _sc_scatter_kernel(x_hbm, pos_hbm, xs_hbm, rows_v, idx_v, sem):
    wid = jax.lax.axis_index("s") * _NC + jax.lax.axis_index("c")
    grp = wid // 8
    h = wid % 8
    src_base = (grp // 2) * _T + h * _PER_W
    aw = wid * _PER_W
    for c in range(_NCHUNK):
        pltpu.sync_copy(x_hbm.at[pl.ds(src_base + c * _CHUNK, _CHUNK)], rows_v)
        pltpu.sync_copy(pos_hbm.at[pl.ds(aw + c * _CHUNK, _CHUNK)], idx_v)
        pltpu.async_copy(rows_v, xs_hbm.at[idx_v], sem).wait()


def _sc_gather_kernel(y_hbm, pos_hbm, yg_hbm, rows_v, idx_v, sem):
    wid = jax.lax.axis_index("s") * _NC + jax.lax.axis_index("c")
    aw = wid * _PER_W
    for c in range(_NCHUNK):
        pltpu.sync_copy(pos_hbm.at[pl.ds(aw + c * _CHUNK, _CHUNK)], idx_v)
        pltpu.async_copy(y_hbm.at[idx_v], rows_v, sem).wait()
        pltpu.sync_copy(rows_v, yg_hbm.at[pl.ds(aw + c * _CHUNK, _CHUNK)])


def _gmm_kernel(s_ref, xs_ref, we_ref, y_ref):
    j = pl.program_id(0)
    lj = jax.lax.rem(j, _NT)
    tt = s_ref[2 * _NT + jax.lax.div(j, _NT)]

    @pl.when(lj < tt)
    def _():
        e = s_ref[j]
        w = we_ref[e]
        y_ref[...] = jax.lax.dot_general(
            xs_ref[...], w, (((1,), (1,)), ((), ())),
            preferred_element_type=jnp.float32,
        )


def _combine_kernel(g_ref, a_ref, b_ref, o_ref):
    g = g_ref[0]
    o_ref[0] = g[:, 0:1] * a_ref[0] + g[:, 1:2] * b_ref[0]


def kernel(x0, x1, Wg0, Wg1, We):
    xs2 = jnp.stack([x0, x1])
    wgs = jnp.stack([Wg0, Wg1])

    g01, p0, p1, meta = pl.pallas_call(
        _route_kernel,
        grid=(2,),
        in_specs=[
            pl.BlockSpec((1, _T, _D), lambda t: (t, 0, 0)),
            pl.BlockSpec((1, _E, _D), lambda t: (t, 0, 0)),
        ],
        out_specs=[
            pl.BlockSpec((1, _T, _E), lambda t: (t, 0, 0)),
            pl.BlockSpec((1, _T, _E), lambda t: (t, 0, 0)),
            pl.BlockSpec((1, _T, _E), lambda t: (t, 0, 0)),
            pl.BlockSpec((1, 2 * _NT, 128), lambda t: (t, 0, 0)),
        ],
        out_shape=[
            jax.ShapeDtypeStruct((2, _T, _E), jnp.float32),
            jax.ShapeDtypeStruct((2, _T, _E), jnp.int32),
            jax.ShapeDtypeStruct((2, _T, _E), jnp.int32),
            jax.ShapeDtypeStruct((2, 2 * _NT, 128), jnp.int32),
        ],
    )(xs2, wgs)

    pos = jnp.concatenate(
        [p0[0, :, 0], p1[0, :, 0], p0[1, :, 0], p1[1, :, 0]]
    )
    meta_arr = jnp.concatenate(
        [meta[0, :_NT, 0], meta[1, :_NT, 0], meta[0, _NT, 0:1], meta[1, _NT, 0:1]]
    )

    x_flat = xs2.reshape(2 * _T, _D)

    mesh = plsc.VectorSubcoreMesh(
        core_axis_name="c", subcore_axis_name="s", num_cores=_NC, num_subcores=_NS
    )
    scatter = functools.partial(
        pl.kernel,
        out_type=jax.ShapeDtypeStruct((2 * _XS, _D), jnp.float32),
        mesh=mesh,
        scratch_types=[
            pltpu.VMEM((_CHUNK, _D), jnp.float32),
            pltpu.VMEM((_CHUNK,), jnp.int32),
            pltpu.SemaphoreType.DMA,
        ],
    )(_sc_scatter_kernel)
    xs_sorted = scatter(x_flat, pos)

    y_full = pl.pallas_call(
        _gmm_kernel,
        grid_spec=pltpu.PrefetchScalarGridSpec(
            num_scalar_prefetch=1,
            grid=(2 * _NT,),
            in_specs=[
                pl.BlockSpec((_TB, _D), lambda j, s: (j, 0)),
                pl.BlockSpec((_E, _D, _D), lambda j, s: (0, 0, 0)),
            ],
            out_specs=pl.BlockSpec((_TB, _D), lambda j, s: (j, 0)),
        ),
        out_shape=jax.ShapeDtypeStruct((2 * _XS, _D), jnp.float32),
    )(meta_arr, xs_sorted, We)

    gather = functools.partial(
        pl.kernel,
        out_type=jax.ShapeDtypeStruct((2 * _NA, _D), jnp.float32),
        mesh=mesh,
        scratch_types=[
            pltpu.VMEM((_CHUNK, _D), jnp.float32),
            pltpu.VMEM((_CHUNK,), jnp.int32),
            pltpu.SemaphoreType.DMA,
        ],
    )(_sc_gather_kernel)
    yg = gather(y_full, pos)
    yg4 = yg.reshape(4, _T, _D)

    out = pl.pallas_call(
        _combine_kernel,
        grid=(2, _T // _TB),
        in_specs=[
            pl.BlockSpec((1, _TB, _E), lambda t, b: (t, b, 0)),
            pl.BlockSpec((1, _TB, _D), lambda t, b: (2 * t, b, 0)),
            pl.BlockSpec((1, _TB, _D), lambda t, b: (2 * t + 1, b, 0)),
        ],
        out_specs=pl.BlockSpec((1, _TB, _D), lambda t, b: (t, b, 0)),
        out_shape=jax.ShapeDtypeStruct((2, _T, _D), jnp.float32),
    )(g01, yg4, yg4)
    return (out[0], out[1])


# speedup vs baseline: 1.0403x; 1.0403x over previous
import functools

import jax
import jax.numpy as jnp
from jax.experimental import pallas as pl
from jax.experimental.pallas import tpu as pltpu
from jax.experimental.pallas import tpu_sc as plsc

_T = 2048
_D = 1024
_E = 8
_TBC = 512
_GB = 256
_NT = 24
_XS = _NT * _GB
_NA = 2 * _T
_NC = 2
_NS = 16
_NW = _NC * _NS
_CHUNK = 64
_PER_W = (2 * _NA) // _NW
_NCHUNK = _PER_W // _CHUNK


def _shift_down(a, s):
    return jnp.concatenate([jnp.zeros((s, a.shape[1]), a.dtype), a[: a.shape[0] - s]], axis=0)


def _lane_shift(a, s):
    return jnp.concatenate([jnp.zeros((a.shape[0], s), a.dtype), a[:, : a.shape[1] - s]], axis=1)


def _route_one(x, wg, task):
    logits = jax.lax.dot_general(
        x, wg, (((1,), (1,)), ((), ())), preferred_element_type=jnp.float32
    )
    iota = jax.lax.broadcasted_iota(jnp.int32, logits.shape, 1)
    m1 = jnp.max(logits, axis=1, keepdims=True)
    i1 = jnp.min(jnp.where(logits == m1, iota, _E), axis=1, keepdims=True)
    mask1 = iota == i1
    rest = jnp.where(mask1, -jnp.inf, logits)
    m2 = jnp.max(rest, axis=1, keepdims=True)
    i2 = jnp.min(jnp.where(rest == m2, iota, _E), axis=1, keepdims=True)
    mask2 = iota == i2
    tm = jnp.where(mask1 | mask2, logits, 0.0)
    gates = tm / (jnp.sum(tm, axis=1, keepdims=True) + 1e-9)
    g0 = jnp.sum(jnp.where(mask1, gates, 0.0), axis=1, keepdims=True)
    g1 = jnp.sum(jnp.where(mask2, gates, 0.0), axis=1, keepdims=True)
    g01 = jnp.where(iota == 0, g0, 0.0) + jnp.where(iota == 1, g1, 0.0)

    oh0 = mask1.astype(jnp.float32)
    oh1 = mask2.astype(jnp.float32)
    c0i = oh0
    c1i = oh1
    s = 1
    while s < _T:
        c0i = c0i + _shift_down(c0i, s)
        c1i = c1i + _shift_down(c1i, s)
        s *= 2
    cum0 = c0i - oh0
    cum1 = c1i - oh1
    cnt0 = jnp.max(c0i, axis=0, keepdims=True)
    cnt1 = jnp.max(c1i, axis=0, keepdims=True)
    n_e = cnt0 + cnt1
    tiles = jnp.floor((n_e + (_GB - 1)) * (1.0 / _GB))
    ti = tiles
    for sh in (1, 2, 4):
        ti = ti + _lane_shift(ti, sh)
    tile_start = ti - tiles
    total_tiles = ti[:, _E - 1 : _E]
    po = tile_start * float(_GB)

    task_off = float(task * _XS)
    pos0 = jnp.sum(oh0 * (po + cum0), axis=1, keepdims=True) + task_off
    pos1 = jnp.sum(oh1 * (po + cnt0 + cum1), axis=1, keepdims=True) + task_off
    ones8 = jnp.ones((1, _E), jnp.float32)
    p0 = (pos0 * ones8).astype(jnp.int32)
    p1 = (pos1 * ones8).astype(jnp.int32)

    ts24 = jnp.broadcast_to(tile_start, (_NT, _E))
    jvec = jax.lax.broadcasted_iota(jnp.int32, (_NT, _E), 0).astype(jnp.float32)
    te = jnp.sum((ts24 <= jvec).astype(jnp.float32), axis=1, keepdims=True) - 1.0
    te = jnp.clip(te, 0.0, float(_E - 1))
    teb = jnp.broadcast_to(te, (_NT, 128))
    ttb = jnp.broadcast_to(total_tiles, (_NT, 128))
    meta = jnp.concatenate([teb, ttb], axis=0).astype(jnp.int32)
    return g01, p0, p1, meta


def _route_kernel(x0_ref, x1_ref, wg0_ref, wg1_ref,
                  g_ref, p0_ref, p1_ref, meta_ref, xc_ref):
    x0 = x0_ref[...]
    x1 = x1_ref[...]
    xc_ref[:_T] = x0
    xc_ref[_T:] = x1
    g01, p0, p1, meta = _route_one(x0, wg0_ref[...], 0)
    g_ref[0], p0_ref[0], p1_ref[0], meta_ref[0] = g01, p0, p1, meta
    g01, p0, p1, meta = _route_one(x1, wg1_ref[...], 1)
    g_ref[1], p0_ref[1], p1_ref[1], meta_ref[1] = g01, p0, p1, meta


def _sc_scatter_kernel(x_hbm, pos_hbm, xs_hbm, rows_v, idx_v, sem):
    wid = jax.lax.axis_index("s") * _NC + jax.lax.axis_index("c")
    grp = wid // 8
    h = wid % 8
    src_base = (grp // 2) * _T + h * _PER_W
    aw = wid * _PER_W
    for c in range(_NCHUNK):
        pltpu.sync_copy(x_hbm.at[pl.ds(src_base + c * _CHUNK, _CHUNK)], rows_v)
        pltpu.sync_copy(pos_hbm.at[pl.ds(aw + c * _CHUNK, _CHUNK)], idx_v)
        pltpu.async_copy(rows_v, xs_hbm.at[idx_v], sem).wait()


def _sc_gather_kernel(y_hbm, pos_hbm, yg_hbm, rows_v, idx_v, sem):
    wid = jax.lax.axis_index("s") * _NC + jax.lax.axis_index("c")
    aw = wid * _PER_W
    for c in range(_NCHUNK):
        pltpu.sync_copy(pos_hbm.at[pl.ds(aw + c * _CHUNK, _CHUNK)], idx_v)
        pltpu.async_copy(y_hbm.at[idx_v], rows_v, sem).wait()
        pltpu.sync_copy(rows_v, yg_hbm.at[pl.ds(aw + c * _CHUNK, _CHUNK)])


def _gmm_kernel(s_ref, xs_ref, we_ref, y_ref):
    j = pl.program_id(0)
    lj = jax.lax.rem(j, _NT)
    tt = s_ref[2 * _NT + jax.lax.div(j, _NT)]

    @pl.when(lj < tt)
    def _():
        e = s_ref[j]
        w = we_ref[e]
        y_ref[...] = jax.lax.dot_general(
            xs_ref[...], w, (((1,), (1,)), ((), ())),
            preferred_element_type=jnp.float32,
        )


def _xs_map(j, s):
    tt = s[2 * _NT + jax.lax.div(j, _NT)]
    lj = jax.lax.rem(j, _NT)
    return (jnp.where(lj < tt, j, j - lj), 0)


def _y_map(j, s):
    tt = s[2 * _NT + jax.lax.div(j, _NT)]
    lj = jax.lax.rem(j, _NT)
    return (jnp.where(lj < tt, j, 2 * _NT), 0)


def _combine_kernel(g_ref, a_ref, b_ref, o_ref):
    g = g_ref[0]
    o_ref[0] = g[:, 0:1] * a_ref[0] + g[:, 1:2] * b_ref[0]


def kernel(x0, x1, Wg0, Wg1, We):
    g01, p0, p1, meta, xcat = pl.pallas_call(
        _route_kernel,
        in_specs=[
            pl.BlockSpec((_T, _D), lambda: (0, 0)),
            pl.BlockSpec((_T, _D), lambda: (0, 0)),
            pl.BlockSpec((_E, _D), lambda: (0, 0)),
            pl.BlockSpec((_E, _D), lambda: (0, 0)),
        ],
        out_specs=[
            pl.BlockSpec((2, _T, _E), lambda: (0, 0, 0)),
            pl.BlockSpec((2, _T, _E), lambda: (0, 0, 0)),
            pl.BlockSpec((2, _T, _E), lambda: (0, 0, 0)),
            pl.BlockSpec((2, 2 * _NT, 128), lambda: (0, 0, 0)),
            pl.BlockSpec((2 * _T, _D), lambda: (0, 0)),
        ],
        out_shape=[
            jax.ShapeDtypeStruct((2, _T, _E), jnp.float32),
            jax.ShapeDtypeStruct((2, _T, _E), jnp.int32),
            jax.ShapeDtypeStruct((2, _T, _E), jnp.int32),
            jax.ShapeDtypeStruct((2, 2 * _NT, 128), jnp.int32),
            jax.ShapeDtypeStruct((2 * _T, _D), jnp.float32),
        ],
    )(x0, x1, Wg0, Wg1)

    pos = jnp.concatenate(
        [p0[0, :, 0], p1[0, :, 0], p0[1, :, 0], p1[1, :, 0]]
    )
    meta_arr = jnp.concatenate(
        [meta[0, :_NT, 0], meta[1, :_NT, 0], meta[0, _NT, 0:1], meta[1, _NT, 0:1]]
    )

    mesh = plsc.VectorSubcoreMesh(
        core_axis_name="c", subcore_axis_name="s", num_cores=_NC, num_subcores=_NS
    )
    scatter = functools.partial(
        pl.kernel,
        out_type=jax.ShapeDtypeStruct((2 * _XS, _D), jnp.float32),
        mesh=mesh,
        scratch_types=[
            pltpu.VMEM((_CHUNK, _D), jnp.float32),
            pltpu.VMEM((_CHUNK,), jnp.int32),
            pltpu.SemaphoreType.DMA,
        ],
    )(_sc_scatter_kernel)
    xs_sorted = scatter(xcat, pos)

    y_full = pl.pallas_call(
        _gmm_kernel,
        grid_spec=pltpu.PrefetchScalarGridSpec(
            num_scalar_prefetch=1,
            grid=(2 * _NT,),
            in_specs=[
                pl.BlockSpec((_GB, _D), _xs_map),
                pl.BlockSpec((_E, _D, _D), lambda j, s: (0, 0, 0)),
            ],
            out_specs=pl.BlockSpec((_GB, _D), _y_map),
        ),
        out_shape=jax.ShapeDtypeStruct((2 * _XS + _GB, _D), jnp.float32),
    )(meta_arr, xs_sorted, We)

    gather = functools.partial(
        pl.kernel,
        out_type=jax.ShapeDtypeStruct((2 * _NA, _D), jnp.float32),
        mesh=mesh,
        scratch_types=[
            pltpu.VMEM((_CHUNK, _D), jnp.float32),
            pltpu.VMEM((_CHUNK,), jnp.int32),
            pltpu.SemaphoreType.DMA,
        ],
    )(_sc_gather_kernel)
    yg = gather(y_full, pos)
    yg4 = yg.reshape(4, _T, _D)

    out = pl.pallas_call(
        _combine_kernel,
        grid=(2, _T // _TBC),
        in_specs=[
            pl.BlockSpec((1, _TBC, _E), lambda t, b: (t, b, 0)),
            pl.BlockSpec((1, _TBC, _D), lambda t, b: (2 * t, b, 0)),
            pl.BlockSpec((1, _TBC, _D), lambda t, b: (2 * t + 1, b, 0)),
        ],
        out_specs=pl.BlockSpec((1, _TBC, _D), lambda t, b: (t, b, 0)),
        out_shape=jax.ShapeDtypeStruct((2, _T, _D), jnp.float32),
    )(g01, yg4, yg4)
    return (out[0], out[1])


# speedup vs baseline: 1.0945x; 1.0521x over previous
import functools

import jax
import jax.numpy as jnp
from jax.experimental import pallas as pl
from jax.experimental.pallas import tpu as pltpu
from jax.experimental.pallas import tpu_sc as plsc

_T = 2048
_D = 1024
_E = 8
_TBC = 512
_GB = 256
_NT = 24
_XS = _NT * _GB
_NA = 2 * _T
_NC = 2
_NS = 16
_NW = _NC * _NS
_CHUNK = 32
_PER_W = _NA // _NW
_NCHUNK = _PER_W // _CHUNK


def _shift_down(a, s):
    return jnp.concatenate([jnp.zeros((s, a.shape[1]), a.dtype), a[: a.shape[0] - s]], axis=0)


def _lane_shift(a, s):
    return jnp.concatenate([jnp.zeros((a.shape[0], s), a.dtype), a[:, : a.shape[1] - s]], axis=1)


def _route_one(x, wg):
    logits = jax.lax.dot_general(
        x, wg, (((1,), (1,)), ((), ())), preferred_element_type=jnp.float32
    )
    iota = jax.lax.broadcasted_iota(jnp.int32, logits.shape, 1)
    m1 = jnp.max(logits, axis=1, keepdims=True)
    i1 = jnp.min(jnp.where(logits == m1, iota, _E), axis=1, keepdims=True)
    mask1 = iota == i1
    rest = jnp.where(mask1, -jnp.inf, logits)
    m2 = jnp.max(rest, axis=1, keepdims=True)
    i2 = jnp.min(jnp.where(rest == m2, iota, _E), axis=1, keepdims=True)
    mask2 = iota == i2
    tm = jnp.where(mask1 | mask2, logits, 0.0)
    gates = tm / (jnp.sum(tm, axis=1, keepdims=True) + 1e-9)
    g0 = jnp.sum(jnp.where(mask1, gates, 0.0), axis=1, keepdims=True)
    g1 = jnp.sum(jnp.where(mask2, gates, 0.0), axis=1, keepdims=True)
    g01 = jnp.where(iota == 0, g0, 0.0) + jnp.where(iota == 1, g1, 0.0)

    oh0 = mask1.astype(jnp.float32)
    oh1 = mask2.astype(jnp.float32)
    c0i = oh0
    c1i = oh1
    s = 1
    while s < _T:
        c0i = c0i + _shift_down(c0i, s)
        c1i = c1i + _shift_down(c1i, s)
        s *= 2
    cum0 = c0i - oh0
    cum1 = c1i - oh1
    cnt0 = jnp.max(c0i, axis=0, keepdims=True)
    cnt1 = jnp.max(c1i, axis=0, keepdims=True)
    n_e = cnt0 + cnt1
    tiles = jnp.floor((n_e + (_GB - 1)) * (1.0 / _GB))
    ti = tiles
    for sh in (1, 2, 4):
        ti = ti + _lane_shift(ti, sh)
    tile_start = ti - tiles
    total_tiles = ti[:, _E - 1 : _E]
    po = tile_start * float(_GB)

    pos0 = jnp.sum(oh0 * (po + cum0), axis=1, keepdims=True)
    pos1 = jnp.sum(oh1 * (po + cnt0 + cum1), axis=1, keepdims=True)
    ones8 = jnp.ones((1, _E), jnp.float32)
    p0 = (pos0 * ones8).astype(jnp.int32)
    p1 = (pos1 * ones8).astype(jnp.int32)

    ts24 = jnp.broadcast_to(tile_start, (_NT, _E))
    jvec = jax.lax.broadcasted_iota(jnp.int32, (_NT, _E), 0).astype(jnp.float32)
    te = jnp.sum((ts24 <= jvec).astype(jnp.float32), axis=1, keepdims=True) - 1.0
    te = jnp.clip(te, 0.0, float(_E - 1))
    teb = jnp.broadcast_to(te, (_NT, 128))
    ttb = jnp.broadcast_to(total_tiles, (_NT, 128))
    meta = jnp.concatenate([teb, ttb], axis=0).astype(jnp.int32)
    return g01, p0, p1, meta


def _route_kernel(x0_ref, x1_ref, wg0_ref, wg1_ref,
                  g_ref, p0_ref, p1_ref, meta_ref):
    g01, p0, p1, meta = _route_one(x0_ref[...], wg0_ref[...])
    g_ref[0], p0_ref[0], p1_ref[0], meta_ref[0] = g01, p0, p1, meta
    g01, p0, p1, meta = _route_one(x1_ref[...], wg1_ref[...])
    g_ref[1], p0_ref[1], p1_ref[1], meta_ref[1] = g01, p0, p1, meta


def _sc_scatter_kernel(x_hbm, pos_hbm, xs_hbm, rows_a, rows_b, idx_a, idx_b,
                       sem_la, sem_lb, sem_sa, sem_sb):
    wid = jax.lax.axis_index("s") * _NC + jax.lax.axis_index("c")
    aw = wid * _PER_W
    src_base = aw % _T
    rows = (rows_a, rows_b)
    idx = (idx_a, idx_b)
    sem_l = (sem_la, sem_lb)
    sem_s = (sem_sa, sem_sb)
    sc = [None, None]
    for c in range(_NCHUNK):
        b = c & 1
        if sc[b] is not None:
            sc[b].wait()
        pltpu.sync_copy(pos_hbm.at[pl.ds(aw + c * _CHUNK, _CHUNK)], idx[b])
        ld = pltpu.async_copy(
            x_hbm.at[pl.ds(src_base + c * _CHUNK, _CHUNK)], rows[b], sem_l[b])
        ld.wait()
        sc[b] = pltpu.async_copy(rows[b], xs_hbm.at[idx[b]], sem_s[b])
    sc[0].wait()
    sc[1].wait()


def _sc_gather_kernel(y_hbm, pos_hbm, yg_hbm, rows_a, rows_b, idx_a, idx_b,
                      sem_la, sem_lb, sem_sa, sem_sb):
    wid = jax.lax.axis_index("s") * _NC + jax.lax.axis_index("c")
    base = wid * _PER_W
    rows = (rows_a, rows_b)
    idx = (idx_a, idx_b)
    sem_l = (sem_la, sem_lb)
    sem_s = (sem_sa, sem_sb)
    st = [None, None]
    for c in range(_NCHUNK):
        b = c & 1
        if st[b] is not None:
            st[b].wait()
        pltpu.sync_copy(pos_hbm.at[pl.ds(base + c * _CHUNK, _CHUNK)], idx[b])
        g = pltpu.async_copy(y_hbm.at[idx[b]], rows[b], sem_l[b])
        g.wait()
        st[b] = pltpu.async_copy(
            rows[b], yg_hbm.at[pl.ds(base + c * _CHUNK, _CHUNK)], sem_s[b])
    st[0].wait()
    st[1].wait()


def _gmm_kernel(s_ref, xs_ref, we_ref, y_ref):
    j = pl.program_id(0)
    tt = s_ref[_NT]

    @pl.when(j < tt)
    def _():
        e = s_ref[j]
        w = we_ref[e]
        y_ref[...] = jax.lax.dot_general(
            xs_ref[...], w, (((1,), (1,)), ((), ())),
            preferred_element_type=jnp.float32,
        )


def _xs_map(j, s):
    return (jnp.where(j < s[_NT], j, 0), 0)


def _y_map(j, s):
    return (jnp.where(j < s[_NT], j, _NT), 0)


def _combine_kernel(g_ref, a_ref, b_ref, o_ref):
    g = g_ref[0]
    o_ref[0] = g[:, 0:1] * a_ref[0] + g[:, 1:2] * b_ref[0]


def _sc_pair(body, out_rows):
    mesh = plsc.VectorSubcoreMesh(
        core_axis_name="c", subcore_axis_name="s", num_cores=_NC, num_subcores=_NS
    )
    return functools.partial(
        pl.kernel,
        out_type=jax.ShapeDtypeStruct((out_rows, _D), jnp.float32),
        mesh=mesh,
        scratch_types=[
            pltpu.VMEM((_CHUNK, _D), jnp.float32),
            pltpu.VMEM((_CHUNK, _D), jnp.float32),
            pltpu.VMEM((_CHUNK,), jnp.int32),
            pltpu.VMEM((_CHUNK,), jnp.int32),
            pltpu.SemaphoreType.DMA,
            pltpu.SemaphoreType.DMA,
            pltpu.SemaphoreType.DMA,
            pltpu.SemaphoreType.DMA,
        ],
    )(body)


def _task_pipeline(x, g01_t, pos_t, meta_t, We):
    xs_sorted = _sc_pair(_sc_scatter_kernel, _XS)(x, pos_t)
    y_full = pl.pallas_call(
        _gmm_kernel,
        grid_spec=pltpu.PrefetchScalarGridSpec(
            num_scalar_prefetch=1,
            grid=(_NT,),
            in_specs=[
                pl.BlockSpec((_GB, _D), _xs_map),
                pl.BlockSpec((_E, _D, _D), lambda j, s: (0, 0, 0)),
            ],
            out_specs=pl.BlockSpec((_GB, _D), _y_map),
        ),
        out_shape=jax.ShapeDtypeStruct((_XS + _GB, _D), jnp.float32),
    )(meta_t, xs_sorted, We)
    yg = _sc_pair(_sc_gather_kernel, _NA)(y_full, pos_t)
    yg2 = yg.reshape(2, _T, _D)
    out = pl.pallas_call(
        _combine_kernel,
        grid=(_T // _TBC,),
        in_specs=[
            pl.BlockSpec((1, _TBC, _E), lambda b: (0, b, 0)),
            pl.BlockSpec((1, _TBC, _D), lambda b: (0, b, 0)),
            pl.BlockSpec((1, _TBC, _D), lambda b: (1, b, 0)),
        ],
        out_specs=pl.BlockSpec((1, _TBC, _D), lambda b: (0, b, 0)),
        out_shape=jax.ShapeDtypeStruct((1, _T, _D), jnp.float32),
    )(g01_t.reshape(1, _T, _E), yg2, yg2)
    return out[0]


def kernel(x0, x1, Wg0, Wg1, We):
    g01, p0, p1, meta = pl.pallas_call(
        _route_kernel,
        in_specs=[
            pl.BlockSpec((_T, _D), lambda: (0, 0)),
            pl.BlockSpec((_T, _D), lambda: (0, 0)),
            pl.BlockSpec((_E, _D), lambda: (0, 0)),
            pl.BlockSpec((_E, _D), lambda: (0, 0)),
        ],
        out_specs=[
            pl.BlockSpec((2, _T, _E), lambda: (0, 0, 0)),
            pl.BlockSpec((2, _T, _E), lambda: (0, 0, 0)),
            pl.BlockSpec((2, _T, _E), lambda: (0, 0, 0)),
            pl.BlockSpec((2, 2 * _NT, 128), lambda: (0, 0, 0)),
        ],
        out_shape=[
            jax.ShapeDtypeStruct((2, _T, _E), jnp.float32),
            jax.ShapeDtypeStruct((2, _T, _E), jnp.int32),
            jax.ShapeDtypeStruct((2, _T, _E), jnp.int32),
            jax.ShapeDtypeStruct((2, 2 * _NT, 128), jnp.int32),
        ],
    )(x0, x1, Wg0, Wg1)

    pos_t0 = jnp.concatenate([p0[0, :, 0], p1[0, :, 0]])
    pos_t1 = jnp.concatenate([p0[1, :, 0], p1[1, :, 0]])
    meta_t0 = jnp.concatenate([meta[0, :_NT, 0], meta[0, _NT, 0:1]])
    meta_t1 = jnp.concatenate([meta[1, :_NT, 0], meta[1, _NT, 0:1]])

    out0 = _task_pipeline(x0, g01[0], pos_t0, meta_t0, We)
    out1 = _task_pipeline(x1, g01[1], pos_t1, meta_t1, We)
    return (out0, out1)


# speedup vs baseline: 1.1063x; 1.0108x over previous
import functools

import jax
import jax.numpy as jnp
from jax.experimental import pallas as pl
from jax.experimental.pallas import tpu as pltpu
from jax.experimental.pallas import tpu_sc as plsc

_T = 2048
_D = 1024
_E = 8
_TBC = 512
_GB = 256
_NT = 24
_XS = _NT * _GB
_NA = 2 * _T
_NC = 2
_NS = 16
_NW = _NC * _NS
_CHUNK = 32
_PER_W = _NA // _NW
_NCHUNK = _PER_W // _CHUNK


def _shift_down(a, s):
    return jnp.concatenate([jnp.zeros((s, a.shape[1]), a.dtype), a[: a.shape[0] - s]], axis=0)


def _lane_shift(a, s):
    return jnp.concatenate([jnp.zeros((a.shape[0], s), a.dtype), a[:, : a.shape[1] - s]], axis=1)


def _route_one(x, wg):
    logits = jax.lax.dot_general(
        x, wg, (((1,), (1,)), ((), ())), preferred_element_type=jnp.float32
    )
    iota = jax.lax.broadcasted_iota(jnp.int32, logits.shape, 1)
    m1 = jnp.max(logits, axis=1, keepdims=True)
    i1 = jnp.min(jnp.where(logits == m1, iota, _E), axis=1, keepdims=True)
    mask1 = iota == i1
    rest = jnp.where(mask1, -jnp.inf, logits)
    m2 = jnp.max(rest, axis=1, keepdims=True)
    i2 = jnp.min(jnp.where(rest == m2, iota, _E), axis=1, keepdims=True)
    mask2 = iota == i2
    tm = jnp.where(mask1 | mask2, logits, 0.0)
    gates = tm / (jnp.sum(tm, axis=1, keepdims=True) + 1e-9)
    g0 = jnp.sum(jnp.where(mask1, gates, 0.0), axis=1, keepdims=True)
    g1 = jnp.sum(jnp.where(mask2, gates, 0.0), axis=1, keepdims=True)
    g01 = jnp.where(iota == 0, g0, 0.0) + jnp.where(iota == 1, g1, 0.0)

    oh0 = mask1.astype(jnp.float32)
    oh1 = mask2.astype(jnp.float32)
    c0i = oh0
    c1i = oh1
    s = 1
    while s < _T:
        c0i = c0i + _shift_down(c0i, s)
        c1i = c1i + _shift_down(c1i, s)
        s *= 2
    cum0 = c0i - oh0
    cum1 = c1i - oh1
    cnt0 = jnp.max(c0i, axis=0, keepdims=True)
    cnt1 = jnp.max(c1i, axis=0, keepdims=True)
    n_e = cnt0 + cnt1
    tiles = jnp.floor((n_e + (_GB - 1)) * (1.0 / _GB))
    ti = tiles
    for sh in (1, 2, 4):
        ti = ti + _lane_shift(ti, sh)
    tile_start = ti - tiles
    total_tiles = ti[:, _E - 1 : _E]
    po = tile_start * float(_GB)

    pos0 = jnp.sum(oh0 * (po + cum0), axis=1, keepdims=True)
    pos1 = jnp.sum(oh1 * (po + cnt0 + cum1), axis=1, keepdims=True)
    ones8 = jnp.ones((1, _E), jnp.float32)
    p0 = (pos0 * ones8).astype(jnp.int32)
    p1 = (pos1 * ones8).astype(jnp.int32)

    ts24 = jnp.broadcast_to(tile_start, (_NT, _E))
    jvec = jax.lax.broadcasted_iota(jnp.int32, (_NT, _E), 0).astype(jnp.float32)
    te = jnp.sum((ts24 <= jvec).astype(jnp.float32), axis=1, keepdims=True) - 1.0
    te = jnp.clip(te, 0.0, float(_E - 1))
    teb = jnp.broadcast_to(te, (_NT, 128))
    ttb = jnp.broadcast_to(total_tiles, (_NT, 128))
    meta = jnp.concatenate([teb, ttb], axis=0).astype(jnp.int32)
    return g01, p0, p1, meta


def _route_kernel(x0_ref, x1_ref, wg0_ref, wg1_ref,
                  g_ref, p0_ref, p1_ref, meta_ref):
    g01, p0, p1, meta = _route_one(x0_ref[...], wg0_ref[...])
    g_ref[0], p0_ref[0], p1_ref[0], meta_ref[0] = g01, p0, p1, meta
    g01, p0, p1, meta = _route_one(x1_ref[...], wg1_ref[...])
    g_ref[1], p0_ref[1], p1_ref[1], meta_ref[1] = g01, p0, p1, meta


def _sc_scatter_kernel(x_hbm, pos_hbm, xs_hbm, rows_a, rows_b, idx_a, idx_b,
                       sem_la, sem_lb, sem_sa, sem_sb):
    wid = jax.lax.axis_index("s") * _NC + jax.lax.axis_index("c")
    aw = wid * _PER_W
    src_base = aw % _T
    rows = (rows_a, rows_b)
    idx = (idx_a, idx_b)
    sem_l = (sem_la, sem_lb)
    sem_s = (sem_sa, sem_sb)
    sc = [None, None]
    for c in range(_NCHUNK):
        b = c & 1
        if sc[b] is not None:
            sc[b].wait()
        pltpu.sync_copy(pos_hbm.at[pl.ds(aw + c * _CHUNK, _CHUNK)], idx[b])
        ld = pltpu.async_copy(
            x_hbm.at[pl.ds(src_base + c * _CHUNK, _CHUNK)], rows[b], sem_l[b])
        ld.wait()
        sc[b] = pltpu.async_copy(rows[b], xs_hbm.at[idx[b]], sem_s[b])
    sc[0].wait()
    sc[1].wait()


def _sc_gather_kernel(y_hbm, pos_hbm, yg_hbm, rows_a, rows_b, idx_a, idx_b,
                      sem_la, sem_lb, sem_sa, sem_sb):
    wid = jax.lax.axis_index("s") * _NC + jax.lax.axis_index("c")
    base = wid * _PER_W
    rows = (rows_a, rows_b)
    idx = (idx_a, idx_b)
    sem_l = (sem_la, sem_lb)
    sem_s = (sem_sa, sem_sb)
    st = [None, None]
    for c in range(_NCHUNK):
        b = c & 1
        if st[b] is not None:
            st[b].wait()
        pltpu.sync_copy(pos_hbm.at[pl.ds(base + c * _CHUNK, _CHUNK)], idx[b])
        g = pltpu.async_copy(y_hbm.at[idx[b]], rows[b], sem_l[b])
        g.wait()
        st[b] = pltpu.async_copy(
            rows[b], yg_hbm.at[pl.ds(base + c * _CHUNK, _CHUNK)], sem_s[b])
    st[0].wait()
    st[1].wait()


def _gmm_kernel(s_ref, xs_ref, we_ref, y_ref):
    j = pl.program_id(0)
    tt = s_ref[_NT]

    @pl.when(j < tt)
    def _():
        y_ref[...] = jax.lax.dot_general(
            xs_ref[...], we_ref[0], (((1,), (1,)), ((), ())),
            preferred_element_type=jnp.float32,
        )


def _xs_map(j, s):
    return (jnp.where(j < s[_NT], j, 0), 0)


def _y_map(j, s):
    return (jnp.where(j < s[_NT], j, _NT), 0)


def _combine_kernel(g_ref, a_ref, b_ref, o_ref):
    g = g_ref[0]
    o_ref[0] = g[:, 0:1] * a_ref[0] + g[:, 1:2] * b_ref[0]


def _sc_pair(body, out_rows):
    mesh = plsc.VectorSubcoreMesh(
        core_axis_name="c", subcore_axis_name="s", num_cores=_NC, num_subcores=_NS
    )
    return functools.partial(
        pl.kernel,
        out_type=jax.ShapeDtypeStruct((out_rows, _D), jnp.float32),
        mesh=mesh,
        scratch_types=[
            pltpu.VMEM((_CHUNK, _D), jnp.float32),
            pltpu.VMEM((_CHUNK, _D), jnp.float32),
            pltpu.VMEM((_CHUNK,), jnp.int32),
            pltpu.VMEM((_CHUNK,), jnp.int32),
            pltpu.SemaphoreType.DMA,
            pltpu.SemaphoreType.DMA,
            pltpu.SemaphoreType.DMA,
            pltpu.SemaphoreType.DMA,
        ],
    )(body)


def _task_pipeline(x, g01_t, pos_t, meta_t, We):
    xs_sorted = _sc_pair(_sc_scatter_kernel, _XS)(x, pos_t)
    y_full = pl.pallas_call(
        _gmm_kernel,
        grid_spec=pltpu.PrefetchScalarGridSpec(
            num_scalar_prefetch=1,
            grid=(_NT,),
            in_specs=[
                pl.BlockSpec((_GB, _D), _xs_map),
                pl.BlockSpec((1, _D, _D), lambda j, s: (s[jnp.minimum(j, s[_NT] - 1)], 0, 0)),
            ],
            out_specs=pl.BlockSpec((_GB, _D), _y_map),
        ),
        out_shape=jax.ShapeDtypeStruct((_XS + _GB, _D), jnp.float32),
    )(meta_t, xs_sorted, We)
    yg = _sc_pair(_sc_gather_kernel, _NA)(y_full, pos_t)
    yg2 = yg.reshape(2, _T, _D)
    out = pl.pallas_call(
        _combine_kernel,
        grid=(_T // _TBC,),
        in_specs=[
            pl.BlockSpec((1, _TBC, _E), lambda b: (0, b, 0)),
            pl.BlockSpec((1, _TBC, _D), lambda b: (0, b, 0)),
            pl.BlockSpec((1, _TBC, _D), lambda b: (1, b, 0)),
        ],
        out_specs=pl.BlockSpec((1, _TBC, _D), lambda b: (0, b, 0)),
        out_shape=jax.ShapeDtypeStruct((1, _T, _D), jnp.float32),
    )(g01_t.reshape(1, _T, _E), yg2, yg2)
    return out[0]


def kernel(x0, x1, Wg0, Wg1, We):
    g01, p0, p1, meta = pl.pallas_call(
        _route_kernel,
        in_specs=[
            pl.BlockSpec((_T, _D), lambda: (0, 0)),
            pl.BlockSpec((_T, _D), lambda: (0, 0)),
            pl.BlockSpec((_E, _D), lambda: (0, 0)),
            pl.BlockSpec((_E, _D), lambda: (0, 0)),
        ],
        out_specs=[
            pl.BlockSpec((2, _T, _E), lambda: (0, 0, 0)),
            pl.BlockSpec((2, _T, _E), lambda: (0, 0, 0)),
            pl.BlockSpec((2, _T, _E), lambda: (0, 0, 0)),
            pl.BlockSpec((2, 2 * _NT, 128), lambda: (0, 0, 0)),
        ],
        out_shape=[
            jax.ShapeDtypeStruct((2, _T, _E), jnp.float32),
            jax.ShapeDtypeStruct((2, _T, _E), jnp.int32),
            jax.ShapeDtypeStruct((2, _T, _E), jnp.int32),
            jax.ShapeDtypeStruct((2, 2 * _NT, 128), jnp.int32),
        ],
    )(x0, x1, Wg0, Wg1)

    pos_t0 = jnp.concatenate([p0[0, :, 0], p1[0, :, 0]])
    pos_t1 = jnp.concatenate([p0[1, :, 0], p1[1, :, 0]])
    meta_t0 = jnp.concatenate([meta[0, :_NT, 0], meta[0, _NT, 0:1]])
    meta_t1 = jnp.concatenate([meta[1, :_NT, 0], meta[1, _NT, 0:1]])

    out0 = _task_pipeline(x0, g01[0], pos_t0, meta_t0, We)
    out1 = _task_pipeline(x1, g01[1], pos_t1, meta_t1, We)
    return (out0, out1)
